# baseline (speedup 1.0000x reference)
import functools

import jax
import jax.numpy as jnp
from jax.experimental import pallas as pl
from jax.experimental.pallas import tpu as pltpu

B, L, C, T = 128, 50, 3, 300
E = 8
K = C * T
D = 768


def _moe_kernel(logits_ref, masks_ref, x_ref, w_ref, b_ref, out_ref, wbf_ref):
    @pl.when(pl.program_id(0) == 0)
    def _():
        wbf_ref[...] = w_ref[...].astype(jnp.bfloat16)

    bs = x_ref.shape[0]
    row0 = pl.program_id(0) * bs
    logits = logits_ref[pl.ds(row0, bs), :]
    mask = (masks_ref[pl.ds(row0, bs), :] == 1).astype(jnp.float32)
    m = jnp.max(logits, axis=1, keepdims=True)
    ex = jnp.exp(logits - m)
    gates = ex / jnp.sum(ex, axis=1, keepdims=True)
    gates = gates * mask
    gates = gates / (jnp.sum(gates, axis=1, keepdims=True) + 1e-9)

    bias = jnp.dot(gates, b_ref[...], preferred_element_type=jnp.float32)
    gates_bf = gates.astype(jnp.bfloat16)

    for i in range(bs):
        acc = gates_bf[i:i + 1, 0:1] * wbf_ref[0]
        for e in range(1, E):
            acc = acc + gates_bf[i:i + 1, e:e + 1] * wbf_ref[e]
        rows = [x_ref[i:i + 1, pl.ds(l * K, K)] for l in range(L)]
        xi = jnp.concatenate(rows, axis=0).astype(jnp.bfloat16)
        out = jnp.dot(xi, acc, preferred_element_type=jnp.float32)
        out_ref[i] = (out + bias[i][None, :]).astype(jnp.bfloat16)


@functools.partial(jax.jit, static_argnames=("bs",))
def _run(x, logits, moe_masks, expert_W, expert_b, bs=8):
    grid = (B // bs,)
    out = pl.pallas_call(
        _moe_kernel,
        grid=grid,
        in_specs=[
            pl.BlockSpec((B, E), lambda i: (0, 0)),
            pl.BlockSpec((B, E), lambda i: (0, 0)),
            pl.BlockSpec((bs, L * K), lambda i: (i, 0)),
            pl.BlockSpec((E, K, D), lambda i: (0, 0, 0)),
            pl.BlockSpec((E, D), lambda i: (0, 0)),
        ],
        out_specs=pl.BlockSpec((bs, L, D), lambda i: (i, 0, 0)),
        out_shape=jax.ShapeDtypeStruct((B, L, D), jnp.bfloat16),
        scratch_shapes=[pltpu.VMEM((E, K, D), jnp.bfloat16)],
    )(logits, moe_masks, x, expert_W, expert_b)
    return out


def kernel(cycle_curve_data, logits, moe_masks, expert_W, expert_b):
    out = _run(cycle_curve_data.reshape(B, L * K), logits,
               moe_masks.astype(jnp.int32), expert_W, expert_b)
    return (out, jnp.float32(0.0))


# speedup vs baseline: 1.1481x; 1.1481x over previous
import functools

import jax
import jax.numpy as jnp
from jax.experimental import pallas as pl
from jax.experimental.pallas import tpu as pltpu

B, L, C, T = 128, 50, 3, 300
E = 8
K = C * T
D = 768


def _moe_kernel(logits_ref, masks_ref, x_ref, w_ref, b_ref, out_ref):
    bs = x_ref.shape[0]
    row0 = pl.program_id(0) * bs
    logits = logits_ref[pl.ds(row0, bs), :]
    mask = (masks_ref[pl.ds(row0, bs), :] == 1).astype(jnp.float32)
    m = jnp.max(logits, axis=1, keepdims=True)
    ex = jnp.exp(logits - m)
    gates = ex / jnp.sum(ex, axis=1, keepdims=True)
    gates = gates * mask
    gates = gates / (jnp.sum(gates, axis=1, keepdims=True) + 1e-9)

    bias = jnp.dot(gates, b_ref[...], preferred_element_type=jnp.float32)
    gates_bf = gates.astype(jnp.bfloat16)

    for i in range(bs):
        acc = gates_bf[i:i + 1, 0:1] * w_ref[0]
        for e in range(1, E):
            acc = acc + gates_bf[i:i + 1, e:e + 1] * w_ref[e]
        xi = x_ref[i].reshape(L, K)
        out = jnp.dot(xi, acc, preferred_element_type=jnp.float32)
        out_ref[i] = (out + bias[i][None, :]).astype(jnp.bfloat16)


@functools.partial(jax.jit, static_argnames=("bs",))
def _run(x, logits, moe_masks, expert_W, expert_b, bs=8):
    grid = (B // bs,)
    xbf = x.astype(jnp.bfloat16)
    wbf = expert_W.astype(jnp.bfloat16)
    out = pl.pallas_call(
        _moe_kernel,
        grid=grid,
        in_specs=[
            pl.BlockSpec((B, E), lambda i: (0, 0)),
            pl.BlockSpec((B, E), lambda i: (0, 0)),
            pl.BlockSpec((bs, L, C, T), lambda i: (i, 0, 0, 0)),
            pl.BlockSpec((E, K, D), lambda i: (0, 0, 0)),
            pl.BlockSpec((E, D), lambda i: (0, 0)),
        ],
        out_specs=pl.BlockSpec((bs, L, D), lambda i: (i, 0, 0)),
        out_shape=jax.ShapeDtypeStruct((B, L, D), jnp.bfloat16),
    )(logits, moe_masks, xbf, wbf, expert_b)
    return out


def kernel(cycle_curve_data, logits, moe_masks, expert_W, expert_b):
    out = _run(cycle_curve_data, logits, moe_masks.astype(jnp.int32),
               expert_W, expert_b)
    return (out, jnp.float32(0.0))
